# baseline (speedup 1.0000x reference)
import functools

import jax
import jax.numpy as jnp
from jax import lax
from jax.experimental import pallas as pl
from jax.experimental.pallas import tpu as pltpu
from jax.experimental.pallas import tpu_sc as plsc

_VOCAB = 100000
_EMBED = 64
_CTX = 200

_NC = 2
_NS = 16
_NW = _NC * _NS
_RPW = 8
_ACTIVE = _CTX // _RPW

_C = 8192
_NB = (_VOCAB + _C - 1) // _C


def _gather_sum_sc(x, W_emb):
  mesh = plsc.VectorSubcoreMesh(core_axis_name="c", subcore_axis_name="s")

  @functools.partial(
      pl.kernel,
      out_type=jax.ShapeDtypeStruct((_NW, _EMBED), jnp.float32),
      mesh=mesh,
      compiler_params=pltpu.CompilerParams(use_tc_tiling_on_sc=False),
      scratch_types=[
          pltpu.VMEM((_RPW,), jnp.int32),
          pltpu.VMEM((_RPW, _EMBED), jnp.float32),
          pltpu.VMEM((_EMBED,), jnp.float32),
          pltpu.SemaphoreType.DMA,
      ],
  )
  def k(w_hbm, x_hbm, out_hbm, idx_v, rows_v, acc_v, sem):
    wid = lax.axis_index("s") * _NC + lax.axis_index("c")

    @pl.when(wid < _ACTIVE)
    def _():
      pltpu.sync_copy(x_hbm.at[pl.ds(wid * _RPW, _RPW)], idx_v)
      pltpu.async_copy(w_hbm.at[idx_v], rows_v, sem).wait()
      for c in range(_EMBED // 16):
        s = rows_v[0, pl.ds(c * 16, 16)]
        for j in range(1, _RPW):
          s = s + rows_v[j, pl.ds(c * 16, 16)]
        acc_v[pl.ds(c * 16, 16)] = s

    @pl.when(wid >= _ACTIVE)
    def _():
      z = jnp.zeros((16,), jnp.float32)
      for c in range(_EMBED // 16):
        acc_v[pl.ds(c * 16, 16)] = z

    pltpu.sync_copy(acc_v, out_hbm.at[wid])

  return k(W_emb, x)


def _matvec_body(part_ref, w_ref, b_ref, h_ref, ctx_ref, m_ref, s_ref):
  i = pl.program_id(0)

  @pl.when(i == 0)
  def _():
    m_ref[0] = -jnp.inf
    s_ref[0] = 0.0

  v = jnp.sum(part_ref[...], axis=0, keepdims=True)
  t = lax.dot_general(v, w_ref[...], (((1,), (1,)), ((), ())),
                      preferred_element_type=jnp.float32)
  t = t + b_ref[...]
  col = i * _C + lax.broadcasted_iota(jnp.int32, (1, _C), 1)
  t = jnp.where(col < _VOCAB, t, -jnp.inf)
  h_ref[...] = t
  m_old = m_ref[0]
  m_new = jnp.maximum(m_old, jnp.max(t))
  s_new = s_ref[0] * jnp.exp(m_old - m_new) + jnp.sum(jnp.exp(t - m_new))
  m_ref[0] = m_new
  s_ref[0] = s_new
  ctx_ref[...] = jnp.broadcast_to(m_new + jnp.log(s_new), (1, 1))


def _matvec_logsumexp(partials, W_lin, b2, interpret=False):
  return pl.pallas_call(
      _matvec_body,
      grid=(_NB,),
      in_specs=[
          pl.BlockSpec((_NW, _EMBED), lambda i: (0, 0)),
          pl.BlockSpec((_C, _EMBED), lambda i: (i, 0)),
          pl.BlockSpec((1, _C), lambda i: (0, i)),
      ],
      out_specs=[
          pl.BlockSpec((1, _C), lambda i: (0, i)),
          pl.BlockSpec((1, 1), lambda i: (0, 0)),
      ],
      out_shape=[
          jax.ShapeDtypeStruct((1, _VOCAB), jnp.float32),
          jax.ShapeDtypeStruct((1, 1), jnp.float32),
      ],
      scratch_shapes=[
          pltpu.SMEM((1,), jnp.float32),
          pltpu.SMEM((1,), jnp.float32),
      ],
      interpret=interpret,
  )(partials, W_lin, b2)


def _normalize_body(h_ref, c_ref, o_ref):
  o_ref[...] = h_ref[...] - c_ref[0, 0]


def _normalize(h, ctx, interpret=False):
  return pl.pallas_call(
      _normalize_body,
      out_shape=jax.ShapeDtypeStruct((1, _VOCAB), jnp.float32),
      interpret=interpret,
  )(h, ctx)


def kernel(x, W_emb, W_lin, b_lin):
  x = x.astype(jnp.int32)
  partials = _gather_sum_sc(x, W_emb)
  h, ctx = _matvec_logsumexp(partials, W_lin, b_lin.reshape(1, _VOCAB))
  return _normalize(h, ctx)


# speedup vs baseline: 5.0957x; 5.0957x over previous
import jax
import jax.numpy as jnp
from jax import lax
from jax.experimental import pallas as pl
from jax.experimental.pallas import tpu as pltpu

_VOCAB = 100000
_EMBED = 64
_CTX = 200

_C = 8192
_NB = (_VOCAB + _C - 1) // _C
_TAIL0 = _VOCAB - 128
_LASTTILE = _VOCAB - 160


def _pass1_body(x_ref, emb_any, tail_ref, w_ref, b_ref, h_ref, ctx_ref,
                gtiles, vbuf, sem, m_ref, s_ref):
  i = pl.program_id(0)

  @pl.when(i == 0)
  def _():
    starts = []
    for j in range(_CTX):
      xj = x_ref[j]
      start = pl.multiple_of(
          jnp.minimum((xj // 128) * 128, _LASTTILE), 128)
      starts.append((xj, start))
      pltpu.make_async_copy(
          emb_any.at[:, pl.ds(start, 128)], gtiles.at[j], sem).start()
    for j in range(_CTX):
      pltpu.make_async_copy(
          emb_any.at[:, pl.ds(0, 128)], gtiles.at[j], sem).wait()
    lane = lax.broadcasted_iota(jnp.int32, (1, 128), 1)
    tail = tail_ref[...]
    acc = jnp.zeros((_EMBED, 128), jnp.float32)
    for j in range(_CTX):
      xj, start = starts[j]
      acc = acc + jnp.where(lane == xj - start, gtiles[j], 0.0)
      in_tail = xj >= _LASTTILE + 128
      acc = acc + jnp.where((lane == xj - _TAIL0) & in_tail, tail, 0.0)
    vbuf[...] = jnp.sum(acc, axis=1, keepdims=True)
    m_ref[0] = -jnp.inf
    s_ref[0] = 0.0

  t = lax.dot_general(vbuf[...], w_ref[...], (((0,), (0,)), ((), ())),
                      preferred_element_type=jnp.float32)
  t = t + b_ref[...]
  col = i * _C + lax.broadcasted_iota(jnp.int32, (1, _C), 1)
  t = jnp.where(col < _VOCAB, t, -jnp.inf)
  h_ref[...] = t
  m_old = m_ref[0]
  m_new = jnp.maximum(m_old, jnp.max(t))
  s_new = s_ref[0] * jnp.exp(m_old - m_new) + jnp.sum(jnp.exp(t - m_new))
  m_ref[0] = m_new
  s_ref[0] = s_new
  ctx_ref[...] = jnp.broadcast_to(m_new + jnp.log(s_new), (1, 1))


def _pass1(x, Wt_emb, emb_tail, Wt_lin, b2, interpret=False):
  return pl.pallas_call(
      _pass1_body,
      grid=(_NB,),
      in_specs=[
          pl.BlockSpec(memory_space=pltpu.SMEM),
          pl.BlockSpec(memory_space=pl.ANY),
          pl.BlockSpec((_EMBED, 128), lambda i: (0, 0)),
          pl.BlockSpec((_EMBED, _C), lambda i: (0, i)),
          pl.BlockSpec((1, _C), lambda i: (0, i)),
      ],
      out_specs=[
          pl.BlockSpec((1, _C), lambda i: (0, i)),
          pl.BlockSpec((1, 1), lambda i: (0, 0)),
      ],
      out_shape=[
          jax.ShapeDtypeStruct((1, _VOCAB), jnp.float32),
          jax.ShapeDtypeStruct((1, 1), jnp.float32),
      ],
      scratch_shapes=[
          pltpu.VMEM((_CTX, _EMBED, 128), jnp.float32),
          pltpu.VMEM((_EMBED, 1), jnp.float32),
          pltpu.SemaphoreType.DMA,
          pltpu.SMEM((1,), jnp.float32),
          pltpu.SMEM((1,), jnp.float32),
      ],
      interpret=interpret,
  )(x, Wt_emb, emb_tail, Wt_lin, b2)


def _normalize_body(h_ref, c_ref, o_ref):
  o_ref[...] = h_ref[...] - c_ref[0, 0]


def _normalize(h, ctx, interpret=False):
  return pl.pallas_call(
      _normalize_body,
      out_shape=jax.ShapeDtypeStruct((1, _VOCAB), jnp.float32),
      interpret=interpret,
  )(h, ctx)


def kernel(x, W_emb, W_lin, b_lin):
  x = x.astype(jnp.int32)
  Wt_emb = W_emb.T
  emb_tail = lax.slice(Wt_emb, (0, _TAIL0), (_EMBED, _VOCAB))
  h, ctx = _pass1(x, Wt_emb, emb_tail, W_lin.T, b_lin.reshape(1, _VOCAB))
  return _normalize(h, ctx)


# speedup vs baseline: 5.6233x; 1.1035x over previous
import jax
import jax.numpy as jnp
from jax import lax
from jax.experimental import pallas as pl
from jax.experimental.pallas import tpu as pltpu

_VOCAB = 100000
_EMBED = 64
_CTX = 200

_C = 8192
_NB = (_VOCAB + _C - 1) // _C
_LASTTILE = _VOCAB - 160
_EDGE = _VOCAB // 128


def _body(x_ref, emb_any, edge_ref, w_ref, b_ref, o_ref,
          gtiles, hbuf, vbuf, sem, m_ref, s_ref):
  p = pl.program_id(0)
  i = pl.program_id(1)

  @pl.when((p == 0) & (i == 0))
  def _():
    starts = []
    for j in range(_CTX):
      xj = x_ref[j]
      start = pl.multiple_of(
          jnp.minimum((xj // 128) * 128, _LASTTILE), 128)
      starts.append((xj, start))
      pltpu.make_async_copy(
          emb_any.at[:, pl.ds(start, 128)], gtiles.at[j], sem).start()
    for j in range(_CTX):
      pltpu.make_async_copy(
          emb_any.at[:, pl.ds(0, 128)], gtiles.at[j], sem).wait()
    lane = lax.broadcasted_iota(jnp.int32, (1, 128), 1)
    edge = edge_ref[...]
    acc = jnp.zeros((_EMBED, 128), jnp.float32)
    for j in range(_CTX):
      xj, start = starts[j]
      acc = acc + jnp.where(lane == xj - start, gtiles[j], 0.0)
      in_tail = xj >= _LASTTILE + 128
      acc = acc + jnp.where((lane == xj - _EDGE * 128) & in_tail, edge, 0.0)
    vbuf[...] = jnp.sum(acc, axis=1, keepdims=True)
    m_ref[0] = -jnp.inf
    s_ref[0] = 0.0

  @pl.when(p == 0)
  def _():
    t = lax.dot_general(vbuf[...], w_ref[...], (((0,), (0,)), ((), ())),
                        preferred_element_type=jnp.float32)
    t = t + jnp.reshape(b_ref[...], (1, _C))
    col = i * _C + lax.broadcasted_iota(jnp.int32, (1, _C), 1)
    t = jnp.where(col < _VOCAB, t, -jnp.inf)
    hbuf[i] = t
    m_old = m_ref[0]
    m_new = jnp.maximum(m_old, jnp.max(t))
    s_new = s_ref[0] * jnp.exp(m_old - m_new) + jnp.sum(jnp.exp(t - m_new))
    m_ref[0] = m_new
    s_ref[0] = s_new

  @pl.when(p == 1)
  def _():
    o_ref[...] = hbuf[i] - (m_ref[0] + jnp.log(s_ref[0]))


def _cbow(x, Wt_emb, Wt_lin, b_lin, interpret=False):
  return pl.pallas_call(
      _body,
      grid=(2, _NB),
      in_specs=[
          pl.BlockSpec(memory_space=pltpu.SMEM),
          pl.BlockSpec(memory_space=pl.ANY),
          pl.BlockSpec((_EMBED, 128), lambda p, i: (0, _EDGE)),
          pl.BlockSpec((_EMBED, _C),
                       lambda p, i: (0, jnp.where(p == 0, i, _NB - 1))),
          pl.BlockSpec((_C,), lambda p, i: (jnp.where(p == 0, i, _NB - 1),)),
      ],
      out_specs=pl.BlockSpec((1, _C), lambda p, i: (0, jnp.where(p == 0, 0, i))),
      out_shape=jax.ShapeDtypeStruct((1, _VOCAB), jnp.float32),
      scratch_shapes=[
          pltpu.VMEM((_CTX, _EMBED, 128), jnp.float32),
          pltpu.VMEM((_NB, 1, _C), jnp.float32),
          pltpu.VMEM((_EMBED, 1), jnp.float32),
          pltpu.SemaphoreType.DMA,
          pltpu.SMEM((1,), jnp.float32),
          pltpu.SMEM((1,), jnp.float32),
      ],
      interpret=interpret,
  )(x, Wt_emb, Wt_emb, Wt_lin, b_lin)


def kernel(x, W_emb, W_lin, b_lin):
  x = x.astype(jnp.int32)
  return _cbow(x, W_emb.T, W_lin.T, b_lin)


# speedup vs baseline: 8.4630x; 1.5050x over previous
import jax
import jax.numpy as jnp
from jax import lax
from jax.experimental import pallas as pl
from jax.experimental.pallas import tpu as pltpu

_VOCAB = 100000
_EMBED = 64
_CTX = 200

_C = 51200
_NB = (_VOCAB + _C - 1) // _C
_LASTTILE = _VOCAB - 160
_EDGE = _VOCAB // 128


def _body(x_ref, emb_any, edge_ref, w_ref, b_ref, o_ref,
          gtiles, hbuf, vbuf, sem, m_ref, s_ref):
  p = pl.program_id(0)
  i = pl.program_id(1)

  @pl.when((p == 0) & (i == 0))
  def _():
    starts = []
    for j in range(_CTX):
      xj = x_ref[j]
      start = pl.multiple_of(
          jnp.minimum((xj // 128) * 128, _LASTTILE), 128)
      starts.append((xj, start))
      pltpu.make_async_copy(
          emb_any.at[:, pl.ds(start, 128)], gtiles.at[j], sem).start()
    for j in range(_CTX):
      pltpu.make_async_copy(
          emb_any.at[:, pl.ds(0, 128)], gtiles.at[j], sem).wait()
    lane = lax.broadcasted_iota(jnp.int32, (1, 128), 1)
    edge = edge_ref[...]
    acc = jnp.zeros((_EMBED, 128), jnp.float32)
    for j in range(_CTX):
      xj, start = starts[j]
      acc = acc + jnp.where(lane == xj - start, gtiles[j], 0.0)
      in_tail = xj >= _LASTTILE + 128
      acc = acc + jnp.where((lane == xj - _EDGE * 128) & in_tail, edge, 0.0)
    vbuf[...] = jnp.sum(acc, axis=1, keepdims=True)
    m_ref[0] = -jnp.inf
    s_ref[0] = 0.0

  @pl.when(p == 0)
  def _():
    t = lax.dot_general(vbuf[...], w_ref[...], (((0,), (0,)), ((), ())),
                        preferred_element_type=jnp.float32)
    t = t + jnp.reshape(b_ref[...], (1, _C))
    col = i * _C + lax.broadcasted_iota(jnp.int32, (1, _C), 1)
    t = jnp.where(col < _VOCAB, t, -jnp.inf)
    hbuf[i] = t
    m_old = m_ref[0]
    m_new = jnp.maximum(m_old, jnp.max(t))
    s_new = s_ref[0] * jnp.exp(m_old - m_new) + jnp.sum(jnp.exp(t - m_new))
    m_ref[0] = m_new
    s_ref[0] = s_new

  @pl.when(p == 1)
  def _():
    o_ref[...] = hbuf[i] - (m_ref[0] + jnp.log(s_ref[0]))


def _cbow(x, Wt_emb, Wt_lin, b_lin, interpret=False):
  return pl.pallas_call(
      _body,
      grid=(2, _NB),
      in_specs=[
          pl.BlockSpec(memory_space=pltpu.SMEM),
          pl.BlockSpec(memory_space=pl.ANY),
          pl.BlockSpec((_EMBED, 128), lambda p, i: (0, _EDGE)),
          pl.BlockSpec((_EMBED, _C),
                       lambda p, i: (0, jnp.where(p == 0, i, _NB - 1))),
          pl.BlockSpec((_C,), lambda p, i: (jnp.where(p == 0, i, _NB - 1),)),
      ],
      out_specs=pl.BlockSpec((1, _C), lambda p, i: (0, jnp.where(p == 0, 0, i))),
      out_shape=jax.ShapeDtypeStruct((1, _VOCAB), jnp.float32),
      scratch_shapes=[
          pltpu.VMEM((_CTX, _EMBED, 128), jnp.float32),
          pltpu.VMEM((_NB, 1, _C), jnp.float32),
          pltpu.VMEM((_EMBED, 1), jnp.float32),
          pltpu.SemaphoreType.DMA,
          pltpu.SMEM((1,), jnp.float32),
          pltpu.SMEM((1,), jnp.float32),
      ],
      interpret=interpret,
  )(x, Wt_emb, Wt_emb, Wt_lin, b_lin)


def kernel(x, W_emb, W_lin, b_lin):
  x = x.astype(jnp.int32)
  return _cbow(x, W_emb.T, W_lin.T, b_lin)
